# baseline (speedup 1.0000x reference)
import functools

import jax
import jax.numpy as jnp
from jax import lax
from jax.experimental import pallas as pl
from jax.experimental.pallas import tpu as pltpu
from jax.experimental.pallas import tpu_sc as plsc

N = 10000
E = 160000
DIN = 256
H = 384
HEADS = 6
CHD = H // HEADS

NPAD = 10240
NC = 2
NSUB = 16
CHK = 128
NCHUNK = 80
EPAD = NSUB * CHK * NCHUNK
PAD_DST = 10200
NHALF = NPAD // 2
DUMP = NHALF - 1

BLK = 1280
GRID = NPAD // BLK

_mesh = plsc.VectorSubcoreMesh(core_axis_name="c", subcore_axis_name="s")


def _zero_vmem(ref, nrow, ncol):
    z = jnp.zeros((16,), jnp.float32)

    def body(r, _):
        for j in range(ncol // 16):
            ref[r, pl.ds(j * 16, 16)] = z
        return 0

    lax.fori_loop(0, nrow, body, 0)


def _halved_idx(idxd_ref, idxq_ref, c):
    for i in range(CHK // 16):
        v = idxd_ref[pl.ds(i * 16, 16)]
        q = v - c * (N // 2)
        ok = (q >= 0) & (q < (N // 2))
        idxq_ref[pl.ds(i * 16, 16)] = jnp.where(ok, q, DUMP)


def _make_deg():
    def body(dstp, zrows, out, idxd, idxq, obuf, acc, sem):
        c = lax.axis_index("c")
        s = lax.axis_index("s")

        lane = lax.broadcasted_iota(jnp.int32, (16,), 0)
        e1 = jnp.where(lane == 0, 1.0, 0.0).astype(jnp.float32)

        def fill(r, _):
            obuf[r, :] = e1
            return 0

        lax.fori_loop(0, CHK, fill, 0)

        for k in range(5):
            pltpu.sync_copy(zrows.at[pl.ds(0, 64)], acc.at[pl.ds(s * 320 + k * 64, 64)])
        plsc.subcore_barrier()

        def chunk(g, _):
            pltpu.sync_copy(dstp.at[pl.ds(s * (CHK * NCHUNK) + g * CHK, CHK)], idxd)
            _halved_idx(idxd, idxq, c)
            pltpu.sync_copy(obuf, acc.at[idxq], add=True)
            return 0

        lax.fori_loop(0, NCHUNK, chunk, 0)
        plsc.subcore_barrier()

        for k in range(5):
            pltpu.sync_copy(acc.at[pl.ds(s * 320 + k * 64, 64)],
                            out.at[c, pl.ds(s * 320 + k * 64, 64)])

    return pl.kernel(
        body,
        out_type=jax.ShapeDtypeStruct((NC, NHALF, 16), jnp.float32),
        mesh=_mesh,
        compiler_params=pltpu.CompilerParams(use_tc_tiling_on_sc=False),
        scratch_types=[
            pltpu.VMEM((CHK,), jnp.int32),
            pltpu.VMEM((CHK,), jnp.int32),
            pltpu.VMEM((CHK, 16), jnp.float32),
            pltpu.VMEM_SHARED((NHALF, 16), jnp.float32),
            pltpu.SemaphoreType.DMA,
        ],
    )


def _make_conv(wq):
    rows_per_tile = NPAD // NSUB

    def body(table4, esd, out, ebuf, idx42, idxd2, rows2, acc,
             semg0, semg1, semg2, semg3, semw0, semw1, semw2, semw3):
        c = lax.axis_index("c")
        s = lax.axis_index("s")
        ebase2 = s * (2 * CHK * NCHUNK)
        semg = (semg0, semg1, semg2, semg3)
        semw = (semw0, semw1, semw2, semw3)
        NQ = NCHUNK // 4

        for p in range(2):

            def load_idx(g, b):
                pltpu.sync_copy(esd.at[pl.ds(ebase2 + g * 2 * CHK, 2 * CHK)],
                                ebuf.at[b])
                for i in range(CHK // 16):
                    sv = ebuf[b, pl.ds(i * 16, 16)]
                    idx42[b, pl.ds(i * 16, 16)] = sv * 4 + 2 * c + p
                    idxd2[b, pl.ds(i * 16, 16)] = ebuf[b, pl.ds(CHK + i * 16, 16)]

            def start_gather(b):
                pltpu.async_copy(table4.at[idx42.at[b]], rows2.at[b], semg[b])

            def wait_gather(b):
                pltpu.make_async_copy(table4.at[idx42.at[b]], rows2.at[b],
                                      semg[b]).wait()

            def start_scatter(b):
                pltpu.async_copy(rows2.at[b], acc.at[idxd2.at[b]], semw[b],
                                 add=True)

            def wait_scatter(b):
                pltpu.make_async_copy(rows2.at[b], acc.at[idxd2.at[b]],
                                      semw[b]).wait()

            _zero_vmem(rows2.at[0], CHK, wq)
            for k in range(rows_per_tile // CHK):
                pltpu.sync_copy(rows2.at[0],
                                acc.at[pl.ds(s * rows_per_tile + k * CHK, CHK)])
            plsc.subcore_barrier()

            for b in range(3):
                load_idx(b, b)
                start_gather(b)

            def quad(i, _):
                for b in range(4):
                    g = 4 * i + b
                    wait_gather(b)
                    start_scatter(b)
                    wait_scatter(b)
                    if b == 0:
                        load_idx(g + 3, 3)
                        start_gather(3)
                    else:
                        @pl.when(i < NQ - 1)
                        def _():
                            load_idx(g + 3, b - 1)
                            start_gather(b - 1)

                return 0

            lax.fori_loop(0, NQ, quad, 0)
            plsc.subcore_barrier()

            for k in range(rows_per_tile // CHK):
                pltpu.sync_copy(acc.at[pl.ds(s * rows_per_tile + k * CHK, CHK)],
                                out.at[2 * c + p, pl.ds(s * rows_per_tile + k * CHK, CHK)])
            plsc.subcore_barrier()

    return pl.kernel(
        body,
        out_type=jax.ShapeDtypeStruct((4, NPAD, wq), jnp.float32),
        mesh=_mesh,
        compiler_params=pltpu.CompilerParams(use_tc_tiling_on_sc=False),
        scratch_types=[
            pltpu.VMEM((4, 2 * CHK), jnp.int32),
            pltpu.VMEM((4, CHK), jnp.int32),
            pltpu.VMEM((4, CHK), jnp.int32),
            pltpu.VMEM((4, CHK, wq), jnp.float32),
            pltpu.VMEM_SHARED((NPAD, wq), jnp.float32),
            pltpu.SemaphoreType.DMA,
            pltpu.SemaphoreType.DMA,
            pltpu.SemaphoreType.DMA,
            pltpu.SemaphoreType.DMA,
            pltpu.SemaphoreType.DMA,
            pltpu.SemaphoreType.DMA,
            pltpu.SemaphoreType.DMA,
            pltpu.SemaphoreType.DMA,
        ],
    )


def _make_conv_single(wh):
    rows_per_tile = NPAD // NSUB

    def body(table2, esd, out, ebuf, idx22, idxd2, rows2, acc,
             semg0, semg1, semg2, semg3, semw0, semw1, semw2, semw3):
        c = lax.axis_index("c")
        s = lax.axis_index("s")
        ebase2 = s * (2 * CHK * NCHUNK)
        semg = (semg0, semg1, semg2, semg3)
        semw = (semw0, semw1, semw2, semw3)
        NQ = NCHUNK // 4

        def load_idx(g, b):
            pltpu.sync_copy(esd.at[pl.ds(ebase2 + g * 2 * CHK, 2 * CHK)],
                            ebuf.at[b])
            for i in range(CHK // 16):
                sv = ebuf[b, pl.ds(i * 16, 16)]
                idx22[b, pl.ds(i * 16, 16)] = sv * 2 + c
                idxd2[b, pl.ds(i * 16, 16)] = ebuf[b, pl.ds(CHK + i * 16, 16)]

        def start_gather(b):
            pltpu.async_copy(table2.at[idx22.at[b]], rows2.at[b], semg[b])

        def wait_gather(b):
            pltpu.make_async_copy(table2.at[idx22.at[b]], rows2.at[b],
                                  semg[b]).wait()

        def start_scatter(b):
            pltpu.async_copy(rows2.at[b], acc.at[idxd2.at[b]], semw[b],
                             add=True)

        def wait_scatter(b):
            pltpu.make_async_copy(rows2.at[b], acc.at[idxd2.at[b]],
                                  semw[b]).wait()

        _zero_vmem(rows2.at[0], CHK, wh)
        for k in range(rows_per_tile // CHK):
            pltpu.sync_copy(rows2.at[0],
                            acc.at[pl.ds(s * rows_per_tile + k * CHK, CHK)])
        plsc.subcore_barrier()

        for b in range(3):
            load_idx(b, b)
            start_gather(b)

        def quad(i, _):
            for b in range(4):
                g = 4 * i + b
                wait_gather(b)
                start_scatter(b)
                wait_scatter(b)
                if b == 0:
                    load_idx(g + 3, 3)
                    start_gather(3)
                else:
                    @pl.when(i < NQ - 1)
                    def _():
                        load_idx(g + 3, b - 1)
                        start_gather(b - 1)

            return 0

        lax.fori_loop(0, NQ, quad, 0)
        plsc.subcore_barrier()

        for k in range(rows_per_tile // CHK):
            pltpu.sync_copy(acc.at[pl.ds(s * rows_per_tile + k * CHK, CHK)],
                            out.at[c, pl.ds(s * rows_per_tile + k * CHK, CHK)])

    return pl.kernel(
        body,
        out_type=jax.ShapeDtypeStruct((NC, NPAD, wh), jnp.float32),
        mesh=_mesh,
        compiler_params=pltpu.CompilerParams(use_tc_tiling_on_sc=False),
        scratch_types=[
            pltpu.VMEM((4, 2 * CHK), jnp.int32),
            pltpu.VMEM((4, CHK), jnp.int32),
            pltpu.VMEM((4, CHK), jnp.int32),
            pltpu.VMEM((4, CHK, wh), jnp.float32),
            pltpu.VMEM_SHARED((NPAD, wh), jnp.float32),
            pltpu.SemaphoreType.DMA,
            pltpu.SemaphoreType.DMA,
            pltpu.SemaphoreType.DMA,
            pltpu.SemaphoreType.DMA,
            pltpu.SemaphoreType.DMA,
            pltpu.SemaphoreType.DMA,
            pltpu.SemaphoreType.DMA,
            pltpu.SemaphoreType.DMA,
        ],
    )


def _make_gat():
    rows_per_tile = NPAD // NSUB

    def body(tgat4, adc, esd, num, den,
             ebuf, idx42, idxd2, idxq2, rows2, adrow2, wrow2, outr2,
             accn, accd, semr0, semr1, sema0, sema1):
        c = lax.axis_index("c")
        s = lax.axis_index("s")

        lane = lax.broadcasted_iota(jnp.int32, (16,), 0)
        head_mask = lane < HEADS
        ebase2 = s * (2 * CHK * NCHUNK)
        semr = (semr0, semr1)
        sema = (sema0, sema1)
        cidx = jnp.minimum(lane + 8, 15).reshape(16, 1)
        gdn = lax.GatherDimensionNumbers(
            offset_dims=(), collapsed_slice_dims=(0,), start_index_map=(0,))

        def bcast_lane(vec, idx_scalar):
            iv = jnp.full((16, 1), idx_scalar, jnp.int32)
            return lax.gather(vec, iv, dimension_numbers=gdn, slice_sizes=(1,),
                              mode=lax.GatherScatterMode.PROMISE_IN_BOUNDS)

        for p in range(2):

            def load_idx(g, b):
                pltpu.sync_copy(esd.at[pl.ds(ebase2 + g * 2 * CHK, 2 * CHK)],
                                ebuf.at[b])
                for i in range(CHK // 16):
                    sv = ebuf[b, pl.ds(i * 16, 16)]
                    idx42[b, pl.ds(i * 16, 16)] = sv * 4 + 2 * c + p
                    dv = ebuf[b, pl.ds(CHK + i * 16, 16)]
                    idxd2[b, pl.ds(i * 16, 16)] = dv
                    q = dv - c * (N // 2)
                    ok = (q >= 0) & (q < (N // 2))
                    idxq2[b, pl.ds(i * 16, 16)] = jnp.where(ok, q, DUMP)

            def start_gathers(b):
                pltpu.async_copy(tgat4.at[idx42.at[b]], rows2.at[b], semr[b])
                pltpu.async_copy(adc.at[idxd2.at[b]], adrow2.at[b], sema[b])

            def wait_gathers(b):
                pltpu.make_async_copy(tgat4.at[idx42.at[b]], rows2.at[b],
                                      semr[b]).wait()
                pltpu.make_async_copy(adc.at[idxd2.at[b]], adrow2.at[b],
                                      sema[b]).wait()

            heads_for_j = [(6 * p + j) // 4 for j in range(6)]

            def compute_edges(b):
                rb = rows2.at[b]
                ab = adrow2.at[b]
                ob = outr2.at[b]
                wb = wrow2.at[b]

                def edge(i4, _):
                    for u in range(4):
                        i = i4 * 4 + u
                        a = rb[i, pl.ds(96, 16)]
                        bv = ab[i, :]
                        e = a + bv
                        l = jnp.where(e > 0, e, e * 0.2)
                        csh = lax.gather(bv, cidx, dimension_numbers=gdn,
                                         slice_sizes=(1,),
                                         mode=lax.GatherScatterMode.PROMISE_IN_BOUNDS)
                        w = jnp.exp(l - csh)
                        w = jnp.where(head_mask, w, 0.0)
                        if p == 0:
                            wb[i, :] = w
                        hprev = None
                        wk = None
                        for j in range(6):
                            hh = heads_for_j[j]
                            if hh != hprev:
                                wk = bcast_lane(w, 3 * c + hh)
                                hprev = hh
                            ob[i, pl.ds(j * 16, 16)] = rb[i, pl.ds(j * 16, 16)] * wk
                    return 0

                lax.fori_loop(0, CHK // 4, edge, 0)

            def scatters(b):
                pltpu.sync_copy(outr2.at[b], accn.at[idxd2.at[b]], add=True)
                if p == 0:
                    pltpu.sync_copy(wrow2.at[b], accd.at[idxq2.at[b]], add=True)

            _zero_vmem(outr2.at[0], CHK, 96)
            for k in range(rows_per_tile // CHK):
                pltpu.sync_copy(outr2.at[0],
                                accn.at[pl.ds(s * rows_per_tile + k * CHK, CHK)])
            if p == 0:
                _zero_vmem(adrow2.at[0], CHK, 16)
                for k in range(5):
                    pltpu.sync_copy(adrow2.at[0, pl.ds(0, 64)],
                                    accd.at[pl.ds(s * 320 + k * 64, 64)])
            plsc.subcore_barrier()

            load_idx(0, 0)
            start_gathers(0)

            def pair(i, _):
                load_idx(2 * i + 1, 1)
                start_gathers(1)
                wait_gathers(0)
                compute_edges(0)
                scatters(0)

                @pl.when(i < NCHUNK // 2 - 1)
                def _():
                    load_idx(2 * i + 2, 0)
                    start_gathers(0)

                wait_gathers(1)
                compute_edges(1)
                scatters(1)
                return 0

            lax.fori_loop(0, NCHUNK // 2, pair, 0)
            plsc.subcore_barrier()

            for k in range(rows_per_tile // CHK):
                pltpu.sync_copy(accn.at[pl.ds(s * rows_per_tile + k * CHK, CHK)],
                                num.at[2 * c + p, pl.ds(s * rows_per_tile + k * CHK, CHK)])
            if p == 0:
                for k in range(5):
                    pltpu.sync_copy(accd.at[pl.ds(s * 320 + k * 64, 64)],
                                    den.at[c, pl.ds(s * 320 + k * 64, 64)])
            plsc.subcore_barrier()

    return pl.kernel(
        body,
        out_type=(jax.ShapeDtypeStruct((4, NPAD, 96), jnp.float32),
                  jax.ShapeDtypeStruct((NC, NHALF, 16), jnp.float32)),
        mesh=_mesh,
        compiler_params=pltpu.CompilerParams(use_tc_tiling_on_sc=False),
        scratch_types=[
            pltpu.VMEM((2, 2 * CHK), jnp.int32),
            pltpu.VMEM((2, CHK), jnp.int32),
            pltpu.VMEM((2, CHK), jnp.int32),
            pltpu.VMEM((2, CHK), jnp.int32),
            pltpu.VMEM((2, CHK, 112), jnp.float32),
            pltpu.VMEM((2, CHK, 16), jnp.float32),
            pltpu.VMEM((2, CHK, 16), jnp.float32),
            pltpu.VMEM((2, CHK, 96), jnp.float32),
            pltpu.VMEM_SHARED((NPAD, 96), jnp.float32),
            pltpu.VMEM_SHARED((NHALF, 16), jnp.float32),
            pltpu.SemaphoreType.DMA,
            pltpu.SemaphoreType.DMA,
            pltpu.SemaphoreType.DMA,
            pltpu.SemaphoreType.DMA,
        ],
    )


def _ln(x, g, b):
    mu = jnp.mean(x, axis=-1, keepdims=True)
    xc = x - mu
    var = jnp.mean(xc * xc, axis=-1, keepdims=True)
    return xc * lax.rsqrt(var + 1e-5) * g + b


def _silu(x):
    return x * jax.nn.sigmoid(x)


def _row_spec(w):
    return pl.BlockSpec((BLK, w), lambda i: (i, 0))


def _full_spec(shape):
    nd = len(shape)
    return pl.BlockSpec(shape, lambda i: (0,) * nd)


def _stk_spec(w):
    return pl.BlockSpec((4, BLK, w), lambda i: (0, i, 0))


def _stage_a(xp, degsel, w0g, b0g, lng, lnb):
    def body(x_ref, d_ref, w_ref, b_ref, g_ref, bb_ref, t0_ref, tg_ref):
        xn = _ln(x_ref[:], g_ref[:], bb_ref[:])
        h = jnp.dot(xn, w_ref[:], preferred_element_type=jnp.float32) + b_ref[:]
        dinv = lax.rsqrt(d_ref[:] + 1.0)
        t0_ref[:] = h[:, :H] * dinv
        tg_ref[:] = h[:, H:] * dinv

    return pl.pallas_call(
        body,
        grid=(GRID,),
        in_specs=[_row_spec(DIN), _row_spec(1), _full_spec((DIN, H + H // 2)),
                  _full_spec((1, H + H // 2)), _full_spec((1, DIN)), _full_spec((1, DIN))],
        out_specs=[_row_spec(H), _row_spec(H // 2)],
        out_shape=[jax.ShapeDtypeStruct((NPAD, H), jnp.float32),
                   jax.ShapeDtypeStruct((NPAD, H // 2), jnp.float32)],
    )(xp, degsel, w0g, b0g, lng, lnb)


def _stage_b(m0, t0, mg, tg, degsel, w1, b0, ln0g, ln0b, bg, lngg, lngb):
    def body(m0_ref, t0_ref, mg_ref, tg_ref, d_ref, w_ref, b0_ref, g0_ref,
             bb0_ref, bg_ref, gg_ref, bbg_ref, x1_ref, t1_ref, xg_ref):
        dinv = lax.rsqrt(d_ref[:] + 1.0)
        m0c = jnp.concatenate([m0_ref[0], m0_ref[1], m0_ref[2], m0_ref[3]], axis=1)
        g0 = dinv * (m0c + t0_ref[:]) + b0_ref[:]
        x1 = _silu(_ln(g0, g0_ref[:], bb0_ref[:]))
        x1_ref[:] = x1
        t1_ref[:] = jnp.dot(x1, w_ref[:], preferred_element_type=jnp.float32) * dinv
        mgc = jnp.concatenate([mg_ref[0], mg_ref[1]], axis=1)
        gg = dinv * (mgc + tg_ref[:]) + bg_ref[:]
        xg_ref[:] = _silu(_ln(gg, gg_ref[:], bbg_ref[:]))

    return pl.pallas_call(
        body,
        grid=(GRID,),
        in_specs=[_stk_spec(H // 4), _row_spec(H), pl.BlockSpec((NC, BLK, H // 4), lambda i: (0, i, 0)), _row_spec(H // 2),
                  _row_spec(1), _full_spec((H, H)), _full_spec((1, H)), _full_spec((1, H)),
                  _full_spec((1, H)), _full_spec((1, H // 2)), _full_spec((1, H // 2)),
                  _full_spec((1, H // 2))],
        out_specs=[_row_spec(H), _row_spec(H), _row_spec(H // 2)],
        out_shape=[jax.ShapeDtypeStruct((NPAD, H), jnp.float32),
                   jax.ShapeDtypeStruct((NPAD, H), jnp.float32),
                   jax.ShapeDtypeStruct((NPAD, H // 2), jnp.float32)],
    )(m0, t0, mg, tg, degsel, w1, b0, ln0g, ln0b, bg, lngg, lngb)


def _stage_c(m, t, xres, degsel, wnext, b, lng, lnb):
    def body(m_ref, t_ref, xr_ref, d_ref, w_ref, b_ref, g_ref, bb_ref,
             x_ref, tn_ref):
        dinv = lax.rsqrt(d_ref[:] + 1.0)
        mc = jnp.concatenate([m_ref[0], m_ref[1], m_ref[2], m_ref[3]], axis=1)
        g = dinv * (mc + t_ref[:]) + b_ref[:]
        x = _silu(_ln(g, g_ref[:], bb_ref[:])) + xr_ref[:]
        x_ref[:] = x
        tn_ref[:] = jnp.dot(x, w_ref[:], preferred_element_type=jnp.float32) * dinv

    return pl.pallas_call(
        body,
        grid=(GRID,),
        in_specs=[_stk_spec(H // 4), _row_spec(H), _row_spec(H), _row_spec(1),
                  _full_spec((H, H)), _full_spec((1, H)), _full_spec((1, H)),
                  _full_spec((1, H))],
        out_specs=[_row_spec(H), _row_spec(H)],
        out_shape=[jax.ShapeDtypeStruct((NPAD, H), jnp.float32),
                   jax.ShapeDtypeStruct((NPAD, H), jnp.float32)],
    )(m, t, xres, degsel, wnext, b, lng, lnb)


def _stage_d(m2, t2, x2, degsel, wa, as8, ad8, b2, ln2g, ln2b):
    def body(m_ref, t_ref, xr_ref, d_ref, wa_ref, as_ref, ad_ref, b_ref,
             g_ref, bb_ref, x3_ref, ha_ref, tg_ref, adt_ref):
        dinv = lax.rsqrt(d_ref[:] + 1.0)
        mc = jnp.concatenate([m_ref[0], m_ref[1], m_ref[2], m_ref[3]], axis=1)
        g = dinv * (mc + t_ref[:]) + b_ref[:]
        x3 = _silu(_ln(g, g_ref[:], bb_ref[:])) + xr_ref[:]
        x3_ref[:] = x3
        ha = jnp.dot(x3, wa_ref[:], preferred_element_type=jnp.float32)
        ha_ref[:] = ha
        asv = jnp.dot(ha, as_ref[:], preferred_element_type=jnp.float32)
        adv = jnp.dot(ha, ad_ref[:], preferred_element_type=jnp.float32)
        cv = jax.nn.leaky_relu(asv + adv, negative_slope=0.2)
        z8 = jnp.zeros_like(asv)
        as16 = jnp.concatenate([asv, z8], axis=1)
        tg_ref[:] = jnp.concatenate([ha[:, 0:96], as16, ha[:, 96:192], as16,
                                     ha[:, 192:288], as16, ha[:, 288:384], as16], axis=1)
        adt_ref[:] = jnp.concatenate([adv, cv], axis=1)

    return pl.pallas_call(
        body,
        grid=(GRID,),
        in_specs=[_stk_spec(H // 4), _row_spec(H), _row_spec(H), _row_spec(1),
                  _full_spec((H, H)), _full_spec((H, 8)), _full_spec((H, 8)),
                  _full_spec((1, H)), _full_spec((1, H)), _full_spec((1, H))],
        out_specs=[_row_spec(H), _row_spec(H), _row_spec(448), _row_spec(16)],
        out_shape=[jax.ShapeDtypeStruct((NPAD, H), jnp.float32),
                   jax.ShapeDtypeStruct((NPAD, H), jnp.float32),
                   jax.ShapeDtypeStruct((NPAD, 448), jnp.float32),
                   jax.ShapeDtypeStruct((NPAD, 16), jnp.float32)],
    )(m2, t2, x2, degsel, wa, as8, ad8, b2, ln2g, ln2b)


def _stage_e(num, den, ha, x3, xg, ba, ln3g, ln3b, wf, bf, lnfg, lnfb,
             wm1, bm1, lnmg, lnmb, wm2p, bm2p):
    def body(num_ref, den_ref, ha_ref, x3_ref, xg_ref, ba_ref, g3_ref, b3_ref,
             wf_ref, bf_ref, gf_ref, bbf_ref, wm1_ref, bm1_ref, gm_ref, bbm_ref,
             wm2_ref, bm2_ref, out_ref):
        numc = jnp.concatenate([num_ref[0], num_ref[1], num_ref[2], num_ref[3]], axis=1) + ha_ref[:]
        den6 = den_ref[:, :HEADS] + (1.0 + 1e-16)
        den384 = jnp.broadcast_to(den6[:, :, None], (BLK, HEADS, CHD)).reshape(BLK, H)
        og = numc / den384 + ba_ref[:]
        x4 = _silu(_ln(og, g3_ref[:], b3_ref[:])) + x3_ref[:]
        xc = jnp.concatenate([x4, xg_ref[:]], axis=1)
        xf = _silu(_ln(jnp.dot(xc, wf_ref[:], preferred_element_type=jnp.float32)
                       + bf_ref[:], gf_ref[:], bbf_ref[:]))
        hm = _silu(_ln(jnp.dot(xf, wm1_ref[:], preferred_element_type=jnp.float32)
                       + bm1_ref[:], gm_ref[:], bbm_ref[:]))
        out_ref[:] = jnp.dot(hm, wm2_ref[:], preferred_element_type=jnp.float32) + bm2_ref[:]

    return pl.pallas_call(
        body,
        grid=(GRID,),
        in_specs=[_stk_spec(H // 4), _row_spec(16), _row_spec(H), _row_spec(H),
                  _row_spec(H // 2), _full_spec((1, H)), _full_spec((1, H)),
                  _full_spec((1, H)), _full_spec((H + H // 2, H)), _full_spec((1, H)),
                  _full_spec((1, H)), _full_spec((1, H)), _full_spec((H, H // 2)),
                  _full_spec((1, H // 2)), _full_spec((1, H // 2)), _full_spec((1, H // 2)),
                  _full_spec((H // 2, 128)), _full_spec((1, 128))],
        out_specs=[_row_spec(128)],
        out_shape=[jax.ShapeDtypeStruct((NPAD, 128), jnp.float32)],
    )(num, den, ha, x3, xg, ba, ln3g, ln3b, wf, bf, lnfg, lnfb,
      wm1, bm1, lnmg, lnmb, wm2p, bm2p)


_deg_kernel = _make_deg()
_conv384 = _make_conv(H // 4)
_convg = _make_conv_single(H // 4)
_gat_kernel = _make_gat()


def kernel(x, edge_index, ln_in_g, ln_in_b, Wg, bg, lng_g, lng_b, W0, b0,
           ln0_g, ln0_b, W1, b1, ln1_g, ln1_b, W2, b2, ln2_g, ln2_b, Wa,
           att_src, att_dst, ba, ln3_g, ln3_b, Wf, bf, lnf_g, lnf_b, Wm1,
           bm1, lnm_g, lnm_b, Wm2, bm2):
    f32 = jnp.float32
    src = edge_index[0]
    dst = edge_index[1]
    npad_e = EPAD - E
    srcp = jnp.concatenate([src, jnp.zeros((npad_e,), jnp.int32)])
    dstp = jnp.concatenate([dst, jnp.full((npad_e,), PAD_DST, jnp.int32)])
    esd = jnp.stack([srcp.reshape(-1, CHK), dstp.reshape(-1, CHK)],
                    axis=1).reshape(-1)
    xp = jnp.pad(x, ((0, NPAD - N), (0, 0)))

    r1 = lambda v: v.reshape(1, -1)

    zrows = jnp.zeros((64, 16), f32)
    degout = _deg_kernel(dstp, zrows)
    degsel = jnp.concatenate([degout[0, :N // 2, 0:1], degout[1, :N // 2, 0:1],
                              jnp.zeros((NPAD - N, 1), f32)], axis=0)

    w0g = jnp.concatenate([W0, Wg], axis=1)
    b0g = jnp.concatenate([b0, bg]).reshape(1, -1)
    t0, tg = _stage_a(xp, degsel, w0g, b0g, r1(ln_in_g), r1(ln_in_b))

    m0 = _conv384(t0.reshape(4 * NPAD, H // 4), esd)
    mg = _convg(tg.reshape(2 * NPAD, H // 4), esd)

    x1, t1, xg = _stage_b(m0, t0, mg, tg, degsel, W1, r1(b0), r1(ln0_g),
                          r1(ln0_b), r1(bg), r1(lng_g), r1(lng_b))

    m1 = _conv384(t1.reshape(4 * NPAD, H // 4), esd)
    x2, t2 = _stage_c(m1, t1, x1, degsel, W2, r1(b1), r1(ln1_g), r1(ln1_b))

    m2 = _conv384(t2.reshape(4 * NPAD, H // 4), esd)
    eye6 = jnp.eye(HEADS, dtype=f32)
    as8 = jnp.pad((att_src[:, None, :] * eye6[:, :, None]).transpose(1, 2, 0)
                  .reshape(H, HEADS), ((0, 0), (0, 2)))
    ad8 = jnp.pad((att_dst[:, None, :] * eye6[:, :, None]).transpose(1, 2, 0)
                  .reshape(H, HEADS), ((0, 0), (0, 2)))
    x3, ha, tgat, adc = _stage_d(m2, t2, x2, degsel, Wa, as8, ad8,
                                 r1(b2), r1(ln2_g), r1(ln2_b))

    num, den = _gat_kernel(tgat.reshape(4 * NPAD, 112), adc, esd)
    denf = jnp.concatenate([den[0, :N // 2], den[1, :N // 2],
                            jnp.zeros((NPAD - N, 16), f32)], axis=0)

    wm2p = jnp.pad(Wm2, ((0, 0), (0, 128 - Wm2.shape[1])))
    bm2p = jnp.pad(bm2, (0, 128 - bm2.shape[0])).reshape(1, -1)
    outp = _stage_e(num, denf, ha, x3, xg, r1(ba), r1(ln3_g), r1(ln3_b),
                    Wf, r1(bf), r1(lnf_g), r1(lnf_b), Wm1, r1(bm1),
                    r1(lnm_g), r1(lnm_b), wm2p, bm2p)
    return outp[0][:N, :Wm2.shape[1]]


# speedup vs baseline: 1.0364x; 1.0364x over previous
import functools

import jax
import jax.numpy as jnp
from jax import lax
from jax.experimental import pallas as pl
from jax.experimental.pallas import tpu as pltpu
from jax.experimental.pallas import tpu_sc as plsc

N = 10000
E = 160000
DIN = 256
H = 384
HEADS = 6
CHD = H // HEADS

NPAD = 10240
NC = 2
NSUB = 16
CHK = 128
NCHUNK = 80
EPAD = NSUB * CHK * NCHUNK
PAD_DST = 10200
NHALF = NPAD // 2
DUMP = NHALF - 1

BLK = 1280
GRID = NPAD // BLK

_mesh = plsc.VectorSubcoreMesh(core_axis_name="c", subcore_axis_name="s")


def _zero_vmem(ref, nrow, ncol):
    z = jnp.zeros((16,), jnp.float32)

    def body(r, _):
        for j in range(ncol // 16):
            ref[r, pl.ds(j * 16, 16)] = z
        return 0

    lax.fori_loop(0, nrow, body, 0)


def _halved_idx(idxd_ref, idxq_ref, c):
    for i in range(CHK // 16):
        v = idxd_ref[pl.ds(i * 16, 16)]
        q = v - c * (N // 2)
        ok = (q >= 0) & (q < (N // 2))
        idxq_ref[pl.ds(i * 16, 16)] = jnp.where(ok, q, DUMP)


def _make_deg():
    def body(dstp, zrows, out, idxd, idxq, obuf, acc, sem):
        c = lax.axis_index("c")
        s = lax.axis_index("s")

        lane = lax.broadcasted_iota(jnp.int32, (16,), 0)
        e1 = jnp.where(lane == 0, 1.0, 0.0).astype(jnp.float32)

        def fill(r, _):
            obuf[r, :] = e1
            return 0

        lax.fori_loop(0, CHK, fill, 0)

        for k in range(5):
            pltpu.sync_copy(zrows.at[pl.ds(0, 64)], acc.at[pl.ds(s * 320 + k * 64, 64)])
        plsc.subcore_barrier()

        def chunk(g, _):
            pltpu.sync_copy(dstp.at[pl.ds(s * (CHK * NCHUNK) + g * CHK, CHK)], idxd)
            _halved_idx(idxd, idxq, c)
            pltpu.sync_copy(obuf, acc.at[idxq], add=True)
            return 0

        lax.fori_loop(0, NCHUNK, chunk, 0)
        plsc.subcore_barrier()

        for k in range(5):
            pltpu.sync_copy(acc.at[pl.ds(s * 320 + k * 64, 64)],
                            out.at[c, pl.ds(s * 320 + k * 64, 64)])

    return pl.kernel(
        body,
        out_type=jax.ShapeDtypeStruct((NC, NHALF, 16), jnp.float32),
        mesh=_mesh,
        compiler_params=pltpu.CompilerParams(use_tc_tiling_on_sc=False),
        scratch_types=[
            pltpu.VMEM((CHK,), jnp.int32),
            pltpu.VMEM((CHK,), jnp.int32),
            pltpu.VMEM((CHK, 16), jnp.float32),
            pltpu.VMEM_SHARED((NHALF, 16), jnp.float32),
            pltpu.SemaphoreType.DMA,
        ],
    )


def _make_conv(wq):
    rows_per_tile = NPAD // NSUB

    def body(table4, esd, out, ebuf, idx42, idxd2, rows2, acc,
             semg0, semg1, semg2, semg3, semw0, semw1, semw2, semw3):
        c = lax.axis_index("c")
        s = lax.axis_index("s")
        ebase2 = s * (2 * CHK * NCHUNK)
        semg = (semg0, semg1, semg2, semg3)
        semw = (semw0, semw1, semw2, semw3)
        NQ = NCHUNK // 4

        for p in range(2):

            def load_idx(g, b):
                pltpu.sync_copy(esd.at[pl.ds(ebase2 + g * 2 * CHK, 2 * CHK)],
                                ebuf.at[b])
                for i in range(CHK // 16):
                    sv = ebuf[b, pl.ds(i * 16, 16)]
                    idx42[b, pl.ds(i * 16, 16)] = sv * 4 + 2 * c + p
                    idxd2[b, pl.ds(i * 16, 16)] = ebuf[b, pl.ds(CHK + i * 16, 16)]

            def start_gather(b):
                pltpu.async_copy(table4.at[idx42.at[b]], rows2.at[b], semg[b])

            def wait_gather(b):
                pltpu.make_async_copy(table4.at[idx42.at[b]], rows2.at[b],
                                      semg[b]).wait()

            def start_scatter(b):
                pltpu.async_copy(rows2.at[b], acc.at[idxd2.at[b]], semw[b],
                                 add=True)

            def wait_scatter(b):
                pltpu.make_async_copy(rows2.at[b], acc.at[idxd2.at[b]],
                                      semw[b]).wait()

            _zero_vmem(rows2.at[0], CHK, wq)
            for k in range(rows_per_tile // CHK):
                pltpu.sync_copy(rows2.at[0],
                                acc.at[pl.ds(s * rows_per_tile + k * CHK, CHK)])
            plsc.subcore_barrier()

            for b in range(3):
                load_idx(b, b)
                start_gather(b)

            def quad(i, _):
                for b in range(4):
                    g = 4 * i + b
                    wait_gather(b)
                    start_scatter(b)
                    wait_scatter(b)
                    if b == 0:
                        load_idx(g + 3, 3)
                        start_gather(3)
                    else:
                        @pl.when(i < NQ - 1)
                        def _():
                            load_idx(g + 3, b - 1)
                            start_gather(b - 1)

                return 0

            lax.fori_loop(0, NQ, quad, 0)
            plsc.subcore_barrier()

            for k in range(rows_per_tile // CHK):
                pltpu.sync_copy(acc.at[pl.ds(s * rows_per_tile + k * CHK, CHK)],
                                out.at[2 * c + p, pl.ds(s * rows_per_tile + k * CHK, CHK)])
            plsc.subcore_barrier()

    return pl.kernel(
        body,
        out_type=jax.ShapeDtypeStruct((4, NPAD, wq), jnp.float32),
        mesh=_mesh,
        compiler_params=pltpu.CompilerParams(use_tc_tiling_on_sc=False),
        scratch_types=[
            pltpu.VMEM((4, 2 * CHK), jnp.int32),
            pltpu.VMEM((4, CHK), jnp.int32),
            pltpu.VMEM((4, CHK), jnp.int32),
            pltpu.VMEM((4, CHK, wq), jnp.float32),
            pltpu.VMEM_SHARED((NPAD, wq), jnp.float32),
            pltpu.SemaphoreType.DMA,
            pltpu.SemaphoreType.DMA,
            pltpu.SemaphoreType.DMA,
            pltpu.SemaphoreType.DMA,
            pltpu.SemaphoreType.DMA,
            pltpu.SemaphoreType.DMA,
            pltpu.SemaphoreType.DMA,
            pltpu.SemaphoreType.DMA,
        ],
    )


def _make_conv_single(wh):
    rows_per_tile = NPAD // NSUB

    def body(table2, esd, out, ebuf, idx22, idxd2, rows2, acc,
             semg0, semg1, semg2, semg3, semw0, semw1, semw2, semw3):
        c = lax.axis_index("c")
        s = lax.axis_index("s")
        ebase2 = s * (2 * CHK * NCHUNK)
        semg = (semg0, semg1, semg2, semg3)
        semw = (semw0, semw1, semw2, semw3)
        NQ = NCHUNK // 4

        def load_idx(g, b):
            pltpu.sync_copy(esd.at[pl.ds(ebase2 + g * 2 * CHK, 2 * CHK)],
                            ebuf.at[b])
            for i in range(CHK // 16):
                sv = ebuf[b, pl.ds(i * 16, 16)]
                idx22[b, pl.ds(i * 16, 16)] = sv * 2 + c
                idxd2[b, pl.ds(i * 16, 16)] = ebuf[b, pl.ds(CHK + i * 16, 16)]

        def start_gather(b):
            pltpu.async_copy(table2.at[idx22.at[b]], rows2.at[b], semg[b])

        def wait_gather(b):
            pltpu.make_async_copy(table2.at[idx22.at[b]], rows2.at[b],
                                  semg[b]).wait()

        def start_scatter(b):
            pltpu.async_copy(rows2.at[b], acc.at[idxd2.at[b]], semw[b],
                             add=True)

        def wait_scatter(b):
            pltpu.make_async_copy(rows2.at[b], acc.at[idxd2.at[b]],
                                  semw[b]).wait()

        _zero_vmem(rows2.at[0], CHK, wh)
        for k in range(rows_per_tile // CHK):
            pltpu.sync_copy(rows2.at[0],
                            acc.at[pl.ds(s * rows_per_tile + k * CHK, CHK)])
        plsc.subcore_barrier()

        for b in range(3):
            load_idx(b, b)
            start_gather(b)

        def quad(i, _):
            for b in range(4):
                g = 4 * i + b
                wait_gather(b)
                start_scatter(b)
                wait_scatter(b)
                if b == 0:
                    load_idx(g + 3, 3)
                    start_gather(3)
                else:
                    @pl.when(i < NQ - 1)
                    def _():
                        load_idx(g + 3, b - 1)
                        start_gather(b - 1)

            return 0

        lax.fori_loop(0, NQ, quad, 0)
        plsc.subcore_barrier()

        for k in range(rows_per_tile // CHK):
            pltpu.sync_copy(acc.at[pl.ds(s * rows_per_tile + k * CHK, CHK)],
                            out.at[c, pl.ds(s * rows_per_tile + k * CHK, CHK)])

    return pl.kernel(
        body,
        out_type=jax.ShapeDtypeStruct((NC, NPAD, wh), jnp.float32),
        mesh=_mesh,
        compiler_params=pltpu.CompilerParams(use_tc_tiling_on_sc=False),
        scratch_types=[
            pltpu.VMEM((4, 2 * CHK), jnp.int32),
            pltpu.VMEM((4, CHK), jnp.int32),
            pltpu.VMEM((4, CHK), jnp.int32),
            pltpu.VMEM((4, CHK, wh), jnp.float32),
            pltpu.VMEM_SHARED((NPAD, wh), jnp.float32),
            pltpu.SemaphoreType.DMA,
            pltpu.SemaphoreType.DMA,
            pltpu.SemaphoreType.DMA,
            pltpu.SemaphoreType.DMA,
            pltpu.SemaphoreType.DMA,
            pltpu.SemaphoreType.DMA,
            pltpu.SemaphoreType.DMA,
            pltpu.SemaphoreType.DMA,
        ],
    )


def _make_gat():
    rows_per_tile = NPAD // NSUB

    def body(tgat4, adc, esd, num, den, wsave,
             ebuf, idx42, idxd2, idxq2, rows2, adrow2, wrow2, outr2,
             accn, accd, semr0, semr1, sema0, sema1):
        c = lax.axis_index("c")
        s = lax.axis_index("s")

        lane = lax.broadcasted_iota(jnp.int32, (16,), 0)
        head_mask = lane < HEADS
        ebase2 = s * (2 * CHK * NCHUNK)
        semr = (semr0, semr1)
        sema = (sema0, sema1)
        cidx = jnp.minimum(lane + 8, 15).reshape(16, 1)
        gdn = lax.GatherDimensionNumbers(
            offset_dims=(), collapsed_slice_dims=(0,), start_index_map=(0,))

        def bcast_lane(vec, idx_scalar):
            iv = jnp.full((16, 1), idx_scalar, jnp.int32)
            return lax.gather(vec, iv, dimension_numbers=gdn, slice_sizes=(1,),
                              mode=lax.GatherScatterMode.PROMISE_IN_BOUNDS)

        for p in range(2):

            def load_idx(g, b):
                pltpu.sync_copy(esd.at[pl.ds(ebase2 + g * 2 * CHK, 2 * CHK)],
                                ebuf.at[b])
                for i in range(CHK // 16):
                    sv = ebuf[b, pl.ds(i * 16, 16)]
                    idx42[b, pl.ds(i * 16, 16)] = sv * 4 + 2 * c + p
                    dv = ebuf[b, pl.ds(CHK + i * 16, 16)]
                    idxd2[b, pl.ds(i * 16, 16)] = dv
                    if p == 0:
                        q = dv - c * (N // 2)
                        ok = (q >= 0) & (q < (N // 2))
                        idxq2[b, pl.ds(i * 16, 16)] = jnp.where(ok, q, DUMP)

            def start_gathers(g, b):
                pltpu.async_copy(tgat4.at[idx42.at[b]], rows2.at[b], semr[b])
                if p == 0:
                    pltpu.async_copy(adc.at[idxd2.at[b]], adrow2.at[b], sema[b])
                else:
                    pltpu.async_copy(
                        wsave.at[c, pl.ds(s * (CHK * NCHUNK) + g * CHK, CHK)],
                        wrow2.at[b], sema[b])

            def wait_gathers(g, b):
                pltpu.make_async_copy(tgat4.at[idx42.at[b]], rows2.at[b],
                                      semr[b]).wait()
                if p == 0:
                    pltpu.make_async_copy(adc.at[idxd2.at[b]], adrow2.at[b],
                                          sema[b]).wait()
                else:
                    pltpu.make_async_copy(
                        wsave.at[c, pl.ds(s * (CHK * NCHUNK) + g * CHK, CHK)],
                        wrow2.at[b], sema[b]).wait()

            heads_for_j = [(6 * p + j) // 4 for j in range(6)]

            def compute_edges(b):
                rb = rows2.at[b]
                ab = adrow2.at[b]
                ob = outr2.at[b]
                wb = wrow2.at[b]

                def edge(i4, _):
                    for u in range(4):
                        i = i4 * 4 + u
                        if p == 0:
                            a = rb[i, pl.ds(96, 16)]
                            bv = ab[i, :]
                            e = a + bv
                            l = jnp.where(e > 0, e, e * 0.2)
                            csh = lax.gather(bv, cidx, dimension_numbers=gdn,
                                             slice_sizes=(1,),
                                             mode=lax.GatherScatterMode.PROMISE_IN_BOUNDS)
                            w = jnp.exp(l - csh)
                            w = jnp.where(head_mask, w, 0.0)
                            wb[i, :] = w
                        else:
                            w = wb[i, :]
                        hprev = None
                        wk = None
                        for j in range(6):
                            hh = heads_for_j[j]
                            if hh != hprev:
                                wk = bcast_lane(w, 3 * c + hh)
                                hprev = hh
                            ob[i, pl.ds(j * 16, 16)] = rb[i, pl.ds(j * 16, 16)] * wk
                    return 0

                lax.fori_loop(0, CHK // 4, edge, 0)

            def scatters(g, b):
                pltpu.sync_copy(outr2.at[b], accn.at[idxd2.at[b]], add=True)
                if p == 0:
                    pltpu.sync_copy(wrow2.at[b], accd.at[idxq2.at[b]], add=True)
                    pltpu.sync_copy(
                        wrow2.at[b],
                        wsave.at[c, pl.ds(s * (CHK * NCHUNK) + g * CHK, CHK)])

            _zero_vmem(outr2.at[0], CHK, 96)
            for k in range(rows_per_tile // CHK):
                pltpu.sync_copy(outr2.at[0],
                                accn.at[pl.ds(s * rows_per_tile + k * CHK, CHK)])
            if p == 0:
                _zero_vmem(adrow2.at[0], CHK, 16)
                for k in range(5):
                    pltpu.sync_copy(adrow2.at[0, pl.ds(0, 64)],
                                    accd.at[pl.ds(s * 320 + k * 64, 64)])
            plsc.subcore_barrier()

            load_idx(0, 0)
            start_gathers(0, 0)

            def pair(i, _):
                load_idx(2 * i + 1, 1)
                start_gathers(2 * i + 1, 1)
                wait_gathers(2 * i, 0)
                compute_edges(0)
                scatters(2 * i, 0)

                @pl.when(i < NCHUNK // 2 - 1)
                def _():
                    load_idx(2 * i + 2, 0)
                    start_gathers(2 * i + 2, 0)

                wait_gathers(2 * i + 1, 1)
                compute_edges(1)
                scatters(2 * i + 1, 1)
                return 0

            lax.fori_loop(0, NCHUNK // 2, pair, 0)
            plsc.subcore_barrier()

            for k in range(rows_per_tile // CHK):
                pltpu.sync_copy(accn.at[pl.ds(s * rows_per_tile + k * CHK, CHK)],
                                num.at[2 * c + p, pl.ds(s * rows_per_tile + k * CHK, CHK)])
            if p == 0:
                for k in range(5):
                    pltpu.sync_copy(accd.at[pl.ds(s * 320 + k * 64, 64)],
                                    den.at[c, pl.ds(s * 320 + k * 64, 64)])
            plsc.subcore_barrier()

    return pl.kernel(
        body,
        out_type=(jax.ShapeDtypeStruct((4, NPAD, 96), jnp.float32),
                  jax.ShapeDtypeStruct((NC, NHALF, 16), jnp.float32),
                  jax.ShapeDtypeStruct((NC, EPAD, 16), jnp.float32)),
        mesh=_mesh,
        compiler_params=pltpu.CompilerParams(use_tc_tiling_on_sc=False),
        scratch_types=[
            pltpu.VMEM((2, 2 * CHK), jnp.int32),
            pltpu.VMEM((2, CHK), jnp.int32),
            pltpu.VMEM((2, CHK), jnp.int32),
            pltpu.VMEM((2, CHK), jnp.int32),
            pltpu.VMEM((2, CHK, 112), jnp.float32),
            pltpu.VMEM((2, CHK, 16), jnp.float32),
            pltpu.VMEM((2, CHK, 16), jnp.float32),
            pltpu.VMEM((2, CHK, 96), jnp.float32),
            pltpu.VMEM_SHARED((NPAD, 96), jnp.float32),
            pltpu.VMEM_SHARED((NHALF, 16), jnp.float32),
            pltpu.SemaphoreType.DMA,
            pltpu.SemaphoreType.DMA,
            pltpu.SemaphoreType.DMA,
            pltpu.SemaphoreType.DMA,
        ],
    )


def _ln(x, g, b):
    mu = jnp.mean(x, axis=-1, keepdims=True)
    xc = x - mu
    var = jnp.mean(xc * xc, axis=-1, keepdims=True)
    return xc * lax.rsqrt(var + 1e-5) * g + b


def _silu(x):
    return x * jax.nn.sigmoid(x)


def _row_spec(w):
    return pl.BlockSpec((BLK, w), lambda i: (i, 0))


def _full_spec(shape):
    nd = len(shape)
    return pl.BlockSpec(shape, lambda i: (0,) * nd)


def _stk_spec(w):
    return pl.BlockSpec((4, BLK, w), lambda i: (0, i, 0))


def _stage_a(xp, degsel, w0g, b0g, lng, lnb):
    def body(x_ref, d_ref, w_ref, b_ref, g_ref, bb_ref, t0_ref, tg_ref):
        xn = _ln(x_ref[:], g_ref[:], bb_ref[:])
        h = jnp.dot(xn, w_ref[:], preferred_element_type=jnp.float32) + b_ref[:]
        dinv = lax.rsqrt(d_ref[:] + 1.0)
        t0_ref[:] = h[:, :H] * dinv
        tg_ref[:] = h[:, H:] * dinv

    return pl.pallas_call(
        body,
        grid=(GRID,),
        in_specs=[_row_spec(DIN), _row_spec(1), _full_spec((DIN, H + H // 2)),
                  _full_spec((1, H + H // 2)), _full_spec((1, DIN)), _full_spec((1, DIN))],
        out_specs=[_row_spec(H), _row_spec(H // 2)],
        out_shape=[jax.ShapeDtypeStruct((NPAD, H), jnp.float32),
                   jax.ShapeDtypeStruct((NPAD, H // 2), jnp.float32)],
    )(xp, degsel, w0g, b0g, lng, lnb)


def _stage_b(m0, t0, mg, tg, degsel, w1, b0, ln0g, ln0b, bg, lngg, lngb):
    def body(m0_ref, t0_ref, mg_ref, tg_ref, d_ref, w_ref, b0_ref, g0_ref,
             bb0_ref, bg_ref, gg_ref, bbg_ref, x1_ref, t1_ref, xg_ref):
        dinv = lax.rsqrt(d_ref[:] + 1.0)
        m0c = jnp.concatenate([m0_ref[0], m0_ref[1], m0_ref[2], m0_ref[3]], axis=1)
        g0 = dinv * (m0c + t0_ref[:]) + b0_ref[:]
        x1 = _silu(_ln(g0, g0_ref[:], bb0_ref[:]))
        x1_ref[:] = x1
        t1_ref[:] = jnp.dot(x1, w_ref[:], preferred_element_type=jnp.float32) * dinv
        mgc = jnp.concatenate([mg_ref[0], mg_ref[1]], axis=1)
        gg = dinv * (mgc + tg_ref[:]) + bg_ref[:]
        xg_ref[:] = _silu(_ln(gg, gg_ref[:], bbg_ref[:]))

    return pl.pallas_call(
        body,
        grid=(GRID,),
        in_specs=[_stk_spec(H // 4), _row_spec(H), pl.BlockSpec((NC, BLK, H // 4), lambda i: (0, i, 0)), _row_spec(H // 2),
                  _row_spec(1), _full_spec((H, H)), _full_spec((1, H)), _full_spec((1, H)),
                  _full_spec((1, H)), _full_spec((1, H // 2)), _full_spec((1, H // 2)),
                  _full_spec((1, H // 2))],
        out_specs=[_row_spec(H), _row_spec(H), _row_spec(H // 2)],
        out_shape=[jax.ShapeDtypeStruct((NPAD, H), jnp.float32),
                   jax.ShapeDtypeStruct((NPAD, H), jnp.float32),
                   jax.ShapeDtypeStruct((NPAD, H // 2), jnp.float32)],
    )(m0, t0, mg, tg, degsel, w1, b0, ln0g, ln0b, bg, lngg, lngb)


def _stage_c(m, t, xres, degsel, wnext, b, lng, lnb):
    def body(m_ref, t_ref, xr_ref, d_ref, w_ref, b_ref, g_ref, bb_ref,
             x_ref, tn_ref):
        dinv = lax.rsqrt(d_ref[:] + 1.0)
        mc = jnp.concatenate([m_ref[0], m_ref[1], m_ref[2], m_ref[3]], axis=1)
        g = dinv * (mc + t_ref[:]) + b_ref[:]
        x = _silu(_ln(g, g_ref[:], bb_ref[:])) + xr_ref[:]
        x_ref[:] = x
        tn_ref[:] = jnp.dot(x, w_ref[:], preferred_element_type=jnp.float32) * dinv

    return pl.pallas_call(
        body,
        grid=(GRID,),
        in_specs=[_stk_spec(H // 4), _row_spec(H), _row_spec(H), _row_spec(1),
                  _full_spec((H, H)), _full_spec((1, H)), _full_spec((1, H)),
                  _full_spec((1, H))],
        out_specs=[_row_spec(H), _row_spec(H)],
        out_shape=[jax.ShapeDtypeStruct((NPAD, H), jnp.float32),
                   jax.ShapeDtypeStruct((NPAD, H), jnp.float32)],
    )(m, t, xres, degsel, wnext, b, lng, lnb)


def _stage_d(m2, t2, x2, degsel, wa, as8, ad8, b2, ln2g, ln2b):
    def body(m_ref, t_ref, xr_ref, d_ref, wa_ref, as_ref, ad_ref, b_ref,
             g_ref, bb_ref, x3_ref, ha_ref, tg_ref, adt_ref):
        dinv = lax.rsqrt(d_ref[:] + 1.0)
        mc = jnp.concatenate([m_ref[0], m_ref[1], m_ref[2], m_ref[3]], axis=1)
        g = dinv * (mc + t_ref[:]) + b_ref[:]
        x3 = _silu(_ln(g, g_ref[:], bb_ref[:])) + xr_ref[:]
        x3_ref[:] = x3
        ha = jnp.dot(x3, wa_ref[:], preferred_element_type=jnp.float32)
        ha_ref[:] = ha
        asv = jnp.dot(ha, as_ref[:], preferred_element_type=jnp.float32)
        adv = jnp.dot(ha, ad_ref[:], preferred_element_type=jnp.float32)
        cv = jax.nn.leaky_relu(asv + adv, negative_slope=0.2)
        z8 = jnp.zeros_like(asv)
        as16 = jnp.concatenate([asv, z8], axis=1)
        tg_ref[:] = jnp.concatenate([ha[:, 0:96], as16, ha[:, 96:192], as16,
                                     ha[:, 192:288], as16, ha[:, 288:384], as16], axis=1)
        adt_ref[:] = jnp.concatenate([adv, cv], axis=1)

    return pl.pallas_call(
        body,
        grid=(GRID,),
        in_specs=[_stk_spec(H // 4), _row_spec(H), _row_spec(H), _row_spec(1),
                  _full_spec((H, H)), _full_spec((H, 8)), _full_spec((H, 8)),
                  _full_spec((1, H)), _full_spec((1, H)), _full_spec((1, H))],
        out_specs=[_row_spec(H), _row_spec(H), _row_spec(448), _row_spec(16)],
        out_shape=[jax.ShapeDtypeStruct((NPAD, H), jnp.float32),
                   jax.ShapeDtypeStruct((NPAD, H), jnp.float32),
                   jax.ShapeDtypeStruct((NPAD, 448), jnp.float32),
                   jax.ShapeDtypeStruct((NPAD, 16), jnp.float32)],
    )(m2, t2, x2, degsel, wa, as8, ad8, b2, ln2g, ln2b)


def _stage_e(num, den, ha, x3, xg, ba, ln3g, ln3b, wf, bf, lnfg, lnfb,
             wm1, bm1, lnmg, lnmb, wm2p, bm2p):
    def body(num_ref, den_ref, ha_ref, x3_ref, xg_ref, ba_ref, g3_ref, b3_ref,
             wf_ref, bf_ref, gf_ref, bbf_ref, wm1_ref, bm1_ref, gm_ref, bbm_ref,
             wm2_ref, bm2_ref, out_ref):
        numc = jnp.concatenate([num_ref[0], num_ref[1], num_ref[2], num_ref[3]], axis=1) + ha_ref[:]
        den6 = den_ref[:, :HEADS] + (1.0 + 1e-16)
        den384 = jnp.broadcast_to(den6[:, :, None], (BLK, HEADS, CHD)).reshape(BLK, H)
        og = numc / den384 + ba_ref[:]
        x4 = _silu(_ln(og, g3_ref[:], b3_ref[:])) + x3_ref[:]
        xc = jnp.concatenate([x4, xg_ref[:]], axis=1)
        xf = _silu(_ln(jnp.dot(xc, wf_ref[:], preferred_element_type=jnp.float32)
                       + bf_ref[:], gf_ref[:], bbf_ref[:]))
        hm = _silu(_ln(jnp.dot(xf, wm1_ref[:], preferred_element_type=jnp.float32)
                       + bm1_ref[:], gm_ref[:], bbm_ref[:]))
        out_ref[:] = jnp.dot(hm, wm2_ref[:], preferred_element_type=jnp.float32) + bm2_ref[:]

    return pl.pallas_call(
        body,
        grid=(GRID,),
        in_specs=[_stk_spec(H // 4), _row_spec(16), _row_spec(H), _row_spec(H),
                  _row_spec(H // 2), _full_spec((1, H)), _full_spec((1, H)),
                  _full_spec((1, H)), _full_spec((H + H // 2, H)), _full_spec((1, H)),
                  _full_spec((1, H)), _full_spec((1, H)), _full_spec((H, H // 2)),
                  _full_spec((1, H // 2)), _full_spec((1, H // 2)), _full_spec((1, H // 2)),
                  _full_spec((H // 2, 128)), _full_spec((1, 128))],
        out_specs=[_row_spec(128)],
        out_shape=[jax.ShapeDtypeStruct((NPAD, 128), jnp.float32)],
    )(num, den, ha, x3, xg, ba, ln3g, ln3b, wf, bf, lnfg, lnfb,
      wm1, bm1, lnmg, lnmb, wm2p, bm2p)


_deg_kernel = _make_deg()
_conv384 = _make_conv(H // 4)
_convg = _make_conv_single(H // 4)
_gat_kernel = _make_gat()


def kernel(x, edge_index, ln_in_g, ln_in_b, Wg, bg, lng_g, lng_b, W0, b0,
           ln0_g, ln0_b, W1, b1, ln1_g, ln1_b, W2, b2, ln2_g, ln2_b, Wa,
           att_src, att_dst, ba, ln3_g, ln3_b, Wf, bf, lnf_g, lnf_b, Wm1,
           bm1, lnm_g, lnm_b, Wm2, bm2):
    f32 = jnp.float32
    src = edge_index[0]
    dst = edge_index[1]
    npad_e = EPAD - E
    srcp = jnp.concatenate([src, jnp.zeros((npad_e,), jnp.int32)])
    dstp = jnp.concatenate([dst, jnp.full((npad_e,), PAD_DST, jnp.int32)])
    esd = jnp.stack([srcp.reshape(-1, CHK), dstp.reshape(-1, CHK)],
                    axis=1).reshape(-1)
    xp = jnp.pad(x, ((0, NPAD - N), (0, 0)))

    r1 = lambda v: v.reshape(1, -1)

    zrows = jnp.zeros((64, 16), f32)
    degout = _deg_kernel(dstp, zrows)
    degsel = jnp.concatenate([degout[0, :N // 2, 0:1], degout[1, :N // 2, 0:1],
                              jnp.zeros((NPAD - N, 1), f32)], axis=0)

    w0g = jnp.concatenate([W0, Wg], axis=1)
    b0g = jnp.concatenate([b0, bg]).reshape(1, -1)
    t0, tg = _stage_a(xp, degsel, w0g, b0g, r1(ln_in_g), r1(ln_in_b))

    m0 = _conv384(t0.reshape(4 * NPAD, H // 4), esd)
    mg = _convg(tg.reshape(2 * NPAD, H // 4), esd)

    x1, t1, xg = _stage_b(m0, t0, mg, tg, degsel, W1, r1(b0), r1(ln0_g),
                          r1(ln0_b), r1(bg), r1(lng_g), r1(lng_b))

    m1 = _conv384(t1.reshape(4 * NPAD, H // 4), esd)
    x2, t2 = _stage_c(m1, t1, x1, degsel, W2, r1(b1), r1(ln1_g), r1(ln1_b))

    m2 = _conv384(t2.reshape(4 * NPAD, H // 4), esd)
    eye6 = jnp.eye(HEADS, dtype=f32)
    as8 = jnp.pad((att_src[:, None, :] * eye6[:, :, None]).transpose(1, 2, 0)
                  .reshape(H, HEADS), ((0, 0), (0, 2)))
    ad8 = jnp.pad((att_dst[:, None, :] * eye6[:, :, None]).transpose(1, 2, 0)
                  .reshape(H, HEADS), ((0, 0), (0, 2)))
    x3, ha, tgat, adc = _stage_d(m2, t2, x2, degsel, Wa, as8, ad8,
                                 r1(b2), r1(ln2_g), r1(ln2_b))

    num, den, _ = _gat_kernel(tgat.reshape(4 * NPAD, 112), adc, esd)
    denf = jnp.concatenate([den[0, :N // 2], den[1, :N // 2],
                            jnp.zeros((NPAD - N, 16), f32)], axis=0)

    wm2p = jnp.pad(Wm2, ((0, 0), (0, 128 - Wm2.shape[1])))
    bm2p = jnp.pad(bm2, (0, 128 - bm2.shape[0])).reshape(1, -1)
    outp = _stage_e(num, denf, ha, x3, xg, r1(ba), r1(ln3_g), r1(ln3_b),
                    Wf, r1(bf), r1(lnf_g), r1(lnf_b), Wm1, r1(bm1),
                    r1(lnm_g), r1(lnm_b), wm2p, bm2p)
    return outp[0][:N, :Wm2.shape[1]]
